# baseline (speedup 1.0000x reference)
import functools

import jax
import jax.numpy as jnp
from jax import lax
from jax.experimental import pallas as pl
from jax.experimental.pallas import tpu as pltpu
from jax.experimental.pallas import tpu_sc as plsc

B = 8192
D = 128
NC = 1
NS = 16
NW = NC * NS
BPW = B // NW
CHUNK = 128
NCHUNK = BPW // CHUNK
LANES = 16
NBUF = 2

_mesh = plsc.VectorSubcoreMesh(core_axis_name="c", subcore_axis_name="s",
                               num_cores=NC)


@functools.partial(
    pl.kernel,
    mesh=_mesh,
    out_type=jax.ShapeDtypeStruct((B,), jnp.float32),
    scratch_types=[
        pltpu.VMEM((NBUF, CHUNK), jnp.int32),
        pltpu.VMEM((NBUF, CHUNK), jnp.int32),
        pltpu.VMEM((NBUF, CHUNK, D), jnp.float32),
        pltpu.VMEM((NBUF, CHUNK, D), jnp.float32),
        pltpu.VMEM((CHUNK,), jnp.float32),
        pltpu.VMEM((LANES, LANES), jnp.float32),
        pltpu.SemaphoreType.DMA,
        pltpu.SemaphoreType.DMA,
    ],
)
def _contrastive_sc(w1_hbm, w2_hbm, e1_hbm, e2_hbm, out_hbm,
                    idx1_v, idx2_v, rows1_v, rows2_v, out_v, tmp_v,
                    sem0, sem1):
    wid = lax.axis_index("s") * NC + lax.axis_index("c")
    base = wid * BPW
    sems = (sem0, sem1)

    def fire(g):
        slot = g % NBUF
        off = base + g * CHUNK
        pltpu.sync_copy(w1_hbm.at[pl.ds(off, CHUNK)], idx1_v.at[slot])
        pltpu.sync_copy(w2_hbm.at[pl.ds(off, CHUNK)], idx2_v.at[slot])
        c1 = pltpu.async_copy(e1_hbm.at[idx1_v.at[slot]], rows1_v.at[slot],
                              sems[slot])
        c2 = pltpu.async_copy(e2_hbm.at[idx2_v.at[slot]], rows2_v.at[slot],
                              sems[slot])
        return c1, c2

    inflight = {0: fire(0)}

    for g in range(NCHUNK):
        slot = g % NBUF
        if g + 1 < NCHUNK:
            inflight[g + 1] = fire(g + 1)
        c1, c2 = inflight.pop(g)
        c1.wait()
        c2.wait()
        r1 = rows1_v.at[slot]
        r2 = rows2_v.at[slot]

        SB = 8

        def group_body(grp, carry, r1=r1, r2=r2):
            def sub_body(sb, c2, r1=r1, r2=r2, grp=grp):
                for k in range(SB):
                    i = grp * LANES + sb * SB + k
                    acc = r1[i, pl.ds(0, LANES)] * r2[i, pl.ds(0, LANES)]
                    for j in range(1, D // LANES):
                        acc = acc + (r1[i, pl.ds(j * LANES, LANES)]
                                     * r2[i, pl.ds(j * LANES, LANES)])
                    tmp_v[sb * SB + k] = acc
                return c2

            lax.fori_loop(0, LANES // SB, sub_body, 0)

            lanes = lax.iota(jnp.int32, LANES)

            def merge(a, b, w):
                low = (lanes % (2 * w)) < w
                perm = jnp.where(low, lanes + w, lanes - w)
                return jnp.where(low, a, jnp.take(b, perm)) + \
                       jnp.where(low, jnp.take(a, perm), b)

            vecs = [tmp_v[k] for k in range(LANES)]
            for w in (8, 4, 2, 1):
                n = len(vecs)
                vecs = [merge(vecs[p], vecs[p + n // 2], w)
                        for p in range(n // 2)]
            tot = vecs[0]
            out_v[pl.ds(grp * LANES, LANES)] = 1.0 / (1.0 + jnp.exp(-tot))
            return carry

        lax.fori_loop(0, CHUNK // LANES, group_body, 0)
        pltpu.sync_copy(out_v, out_hbm.at[pl.ds(base + g * CHUNK, CHUNK)])


def kernel(word1, word2, emb1_weight, emb2_weight):
    h0 = _contrastive_sc(word1[:B], word2[:B], emb1_weight, emb2_weight)
    h1 = _contrastive_sc(word1[B:], word2[B:], emb1_weight, emb2_weight)
    return jnp.concatenate([h0, h1])


# speedup vs baseline: 1.6371x; 1.6371x over previous
import functools

import jax
import jax.numpy as jnp
from jax import lax
from jax.experimental import pallas as pl
from jax.experimental.pallas import tpu as pltpu
from jax.experimental.pallas import tpu_sc as plsc

B = 16384
D = 128
NC = 2
NS = 16
NW = NC * NS
BPW = B // NW
CHUNK = 128
NCHUNK = BPW // CHUNK
LANES = 16
NBUF = 2

_mesh = plsc.VectorSubcoreMesh(core_axis_name="c", subcore_axis_name="s",
                               num_cores=NC)


@functools.partial(
    pl.kernel,
    mesh=_mesh,
    out_type=jax.ShapeDtypeStruct((B,), jnp.float32),
    scratch_types=[
        pltpu.VMEM((NBUF, CHUNK), jnp.int32),
        pltpu.VMEM((NBUF, CHUNK), jnp.int32),
        pltpu.VMEM((NBUF, CHUNK, D), jnp.float32),
        pltpu.VMEM((NBUF, CHUNK, D), jnp.float32),
        pltpu.VMEM((CHUNK,), jnp.float32),
        pltpu.VMEM((2, LANES), jnp.float32),
        pltpu.SemaphoreType.DMA,
        pltpu.SemaphoreType.DMA,
    ],
)
def _contrastive_sc(w1_hbm, w2_hbm, e1_hbm, e2_hbm, out_hbm,
                    idx1_v, idx2_v, rows1_v, rows2_v, out_v, tmp_v,
                    sem0, sem1):
    wid = lax.axis_index("s") * NC + lax.axis_index("c")
    base = wid * BPW
    sems = (sem0, sem1)

    def fire(g):
        slot = g % NBUF
        off = base + g * CHUNK
        pltpu.sync_copy(w1_hbm.at[pl.ds(off, CHUNK)], idx1_v.at[slot])
        pltpu.sync_copy(w2_hbm.at[pl.ds(off, CHUNK)], idx2_v.at[slot])
        c1 = pltpu.async_copy(e1_hbm.at[idx1_v.at[slot]], rows1_v.at[slot],
                              sems[slot])
        c2 = pltpu.async_copy(e2_hbm.at[idx2_v.at[slot]], rows2_v.at[slot],
                              sems[slot])
        return c1, c2

    inflight = {0: fire(0)}

    for g in range(NCHUNK):
        slot = g % NBUF
        if g + 1 < NCHUNK:
            inflight[g + 1] = fire(g + 1)
        c1, c2 = inflight.pop(g)
        c1.wait()
        c2.wait()
        r1 = rows1_v.at[slot]
        r2 = rows2_v.at[slot]

        SB = 8

        def group_body(grp, carry, r1=r1, r2=r2):
            lanes = lax.iota(jnp.int32, LANES)

            def merge(a, b, w):
                low = (lanes % (2 * w)) < w
                perm = jnp.where(low, lanes + w, lanes - w)
                return jnp.where(low, a, jnp.take(b, perm)) + \
                       jnp.where(low, jnp.take(a, perm), b)

            def sub_body(sb, c2, r1=r1, r2=r2, grp=grp):
                vecs = []
                for k in range(SB):
                    i = grp * LANES + 2 * k + sb
                    acc = r1[i, pl.ds(0, LANES)] * r2[i, pl.ds(0, LANES)]
                    for j in range(1, D // LANES):
                        acc = acc + (r1[i, pl.ds(j * LANES, LANES)]
                                     * r2[i, pl.ds(j * LANES, LANES)])
                    vecs.append(acc)
                for w in (8, 4, 2):
                    n = len(vecs)
                    vecs = [merge(vecs[p], vecs[p + n // 2], w)
                            for p in range(n // 2)]
                tmp_v[sb] = vecs[0]
                return c2

            lax.fori_loop(0, LANES // SB, sub_body, 0)

            tot = merge(tmp_v[0], tmp_v[1], 1)
            out_v[pl.ds(grp * LANES, LANES)] = 1.0 / (1.0 + jnp.exp(-tot))
            return carry

        lax.fori_loop(0, CHUNK // LANES, group_body, 0)
        pltpu.sync_copy(out_v, out_hbm.at[pl.ds(base + g * CHUNK, CHUNK)])


def kernel(word1, word2, emb1_weight, emb2_weight):
    return _contrastive_sc(word1, word2, emb1_weight, emb2_weight)


# speedup vs baseline: 1.6461x; 1.0055x over previous
import functools

import jax
import jax.numpy as jnp
from jax import lax
from jax.experimental import pallas as pl
from jax.experimental.pallas import tpu as pltpu
from jax.experimental.pallas import tpu_sc as plsc

B = 16384
D = 128
NC = 2
NS = 16
NW = NC * NS
BPW = B // NW
CHUNK = 128
NCHUNK = BPW // CHUNK
LANES = 16
NBUF = 3

_mesh = plsc.VectorSubcoreMesh(core_axis_name="c", subcore_axis_name="s",
                               num_cores=NC)


@functools.partial(
    pl.kernel,
    mesh=_mesh,
    out_type=jax.ShapeDtypeStruct((B,), jnp.float32),
    scratch_types=[
        pltpu.VMEM((BPW,), jnp.int32),
        pltpu.VMEM((BPW,), jnp.int32),
        pltpu.VMEM((NBUF, CHUNK, D), jnp.float32),
        pltpu.VMEM((NBUF, CHUNK, D), jnp.float32),
        pltpu.VMEM((2, CHUNK), jnp.float32),
        pltpu.VMEM((2, LANES), jnp.float32),
        pltpu.SemaphoreType.DMA,
        pltpu.SemaphoreType.DMA,
        pltpu.SemaphoreType.DMA,
        pltpu.SemaphoreType.DMA,
    ],
)
def _contrastive_sc(w1_hbm, w2_hbm, e1_hbm, e2_hbm, out_hbm,
                    idx1_v, idx2_v, rows1_v, rows2_v, out_vs, tmp_v,
                    sem0, sem1, sem2, osem):
    wid = lax.axis_index("s") * NC + lax.axis_index("c")
    base = wid * BPW
    sems = (sem0, sem1, sem2)
    pltpu.sync_copy(w1_hbm.at[pl.ds(base, BPW)], idx1_v)
    pltpu.sync_copy(w2_hbm.at[pl.ds(base, BPW)], idx2_v)

    def fire(g):
        slot = g % NBUF
        c1 = pltpu.async_copy(
            e1_hbm.at[idx1_v.at[pl.ds(g * CHUNK, CHUNK)]],
            rows1_v.at[slot], sems[slot])
        c2 = pltpu.async_copy(
            e2_hbm.at[idx2_v.at[pl.ds(g * CHUNK, CHUNK)]],
            rows2_v.at[slot], sems[slot])
        return c1, c2

    inflight = {g: fire(g) for g in range(min(NBUF - 1, NCHUNK))}
    outflight = {}

    for g in range(NCHUNK):
        slot = g % NBUF
        if g + NBUF - 1 < NCHUNK:
            inflight[g + NBUF - 1] = fire(g + NBUF - 1)
        c1, c2 = inflight.pop(g)
        c1.wait()
        c2.wait()
        if g >= 2:
            outflight.pop(g - 2).wait()
        r1 = rows1_v.at[slot]
        r2 = rows2_v.at[slot]
        out_v = out_vs.at[g % 2]

        SB = 8

        def group_body(grp, carry, r1=r1, r2=r2):
            lanes = lax.iota(jnp.int32, LANES)

            def merge(a, b, w):
                low = (lanes % (2 * w)) < w
                perm = jnp.where(low, lanes + w, lanes - w)
                return jnp.where(low, a, jnp.take(b, perm)) + \
                       jnp.where(low, jnp.take(a, perm), b)

            def sub_body(sb, c2, r1=r1, r2=r2, grp=grp):
                vecs = []
                for k in range(SB):
                    i = grp * LANES + 2 * k + sb
                    acc = r1[i, pl.ds(0, LANES)] * r2[i, pl.ds(0, LANES)]
                    for j in range(1, D // LANES):
                        acc = acc + (r1[i, pl.ds(j * LANES, LANES)]
                                     * r2[i, pl.ds(j * LANES, LANES)])
                    vecs.append(acc)
                for w in (8, 4, 2):
                    n = len(vecs)
                    vecs = [merge(vecs[p], vecs[p + n // 2], w)
                            for p in range(n // 2)]
                tmp_v[sb] = vecs[0]
                return c2

            lax.fori_loop(0, LANES // SB, sub_body, 0)

            tot = merge(tmp_v[0], tmp_v[1], 1)
            out_v[pl.ds(grp * LANES, LANES)] = 1.0 / (1.0 + jnp.exp(-tot))
            return carry

        lax.fori_loop(0, CHUNK // LANES, group_body, 0)
        outflight[g] = pltpu.async_copy(
            out_v, out_hbm.at[pl.ds(base + g * CHUNK, CHUNK)], osem)

    for g in sorted(outflight):
        outflight.pop(g).wait()


def kernel(word1, word2, emb1_weight, emb2_weight):
    return _contrastive_sc(word1, word2, emb1_weight, emb2_weight)
